# Initial kernel scaffold; baseline (speedup 1.0000x reference)
import jax
import jax.numpy as jnp
from jax.experimental import pallas as pl
from jax.experimental.pallas import tpu as pltpu

_K = 16
_INF = jnp.inf


def _pc_body(xgT_ref, geoc_ref, geot_ref, wT_ref, wg_ref, b_ref, *rest):
    out_ref = rest[-1]
    mlp = rest[:-1]
    xgT = xgT_ref[0]
    geot = geot_ref[0]
    geoc = geoc_ref[0]
    projT = jnp.dot(wT_ref[...], xgT,
                    preferred_element_type=jnp.float32)
    d = None
    for dim in range(3):
        diff = geoc[:, dim:dim + 1] - geot[dim:dim + 1, :]
        sq = diff * diff
        d = sq if d is None else d + sq
    for _ in range(_K):
        m = jnp.min(d, axis=1, keepdims=True)
        d = jnp.where(d == m, _INF, d)
    sel = d == _INF
    cols = []
    for c in range(8):
        mc = jnp.max(jnp.where(sel, projT[c:c + 1, :], -_INF),
                     axis=1, keepdims=True)
        cols.append(mc)
    mx = jnp.concatenate(cols, axis=1)
    wg = wg_ref[...]
    dotc = (geoc[:, 0:1] * wg[0:1, :] + geoc[:, 1:2] * wg[1:2, :]
            + geoc[:, 2:3] * wg[2:3, :])
    h = jnp.maximum(mx - dotc + b_ref[...], 0.0)
    if mlp:
        wm1, bm1, wm2, bm2, wm3, bm3 = mlp
        h = jnp.maximum(
            jnp.dot(h, wm1[...], preferred_element_type=jnp.float32)
            + bm1[...], 0.0)
        h = jnp.maximum(
            jnp.dot(h, wm2[...], preferred_element_type=jnp.float32)
            + bm2[...], 0.0)
        h = jnp.maximum(
            jnp.dot(h, wm3[...], preferred_element_type=jnp.float32)
            + bm3[...], 0.0)
    out_ref[0] = h


def _full_spec(shape):
    nd = len(shape)
    return pl.BlockSpec(shape, lambda b, r, _n=nd: (0,) * _n)


def _point_conv_call(xgT, geoc, geot, wT, wg, b, R, mlp_ws=(),
                     interpret=False):
    B, _, N = xgT.shape
    Nout = geoc.shape[1]
    Fout = 6 if mlp_ws else 8
    grid = (B, Nout // R)
    in_specs = [
        pl.BlockSpec((1, 11, N), lambda b_, r: (b_, 0, 0)),
        pl.BlockSpec((1, R, 3), lambda b_, r: (b_, r, 0)),
        pl.BlockSpec((1, 3, N), lambda b_, r: (b_, 0, 0)),
        _full_spec(wT.shape),
        _full_spec(wg.shape),
        _full_spec(b.shape),
    ]
    args = [xgT, geoc, geot, wT, wg, b]
    for w in mlp_ws:
        in_specs.append(_full_spec(w.shape))
        args.append(w)
    return pl.pallas_call(
        _pc_body,
        grid=grid,
        in_specs=in_specs,
        out_specs=pl.BlockSpec((1, R, Fout), lambda b_, r: (b_, r, 0)),
        out_shape=jax.ShapeDtypeStruct((B, Nout, Fout), jnp.float32),
        compiler_params=pltpu.CompilerParams(
            dimension_semantics=("parallel", "parallel")),
        interpret=interpret,
    )(*args)


def kernel(x, geoin, Wpc1, bpc1, Wpc2, bpc2, Wm1, bm1, Wm2, bm2, Wm3, bm3,
           interpret=False):
    B, Ns, _ = x.shape
    N1, N2 = Ns // 2, Ns // 4
    xg1T = jnp.concatenate([x, geoin], axis=-1).transpose(0, 2, 1)
    geot1 = geoin.transpose(0, 2, 1)
    geoc1 = geoin[:, :N1, :]
    h1 = _point_conv_call(xg1T, geoc1, geot1, Wpc1.T, Wpc1[8:11, :],
                          bpc1[None, :], R=256, interpret=interpret)
    xg2T = jnp.concatenate([h1, geoc1], axis=-1).transpose(0, 2, 1)
    geot2 = geoc1.transpose(0, 2, 1)
    geoc2 = geoin[:, :N2, :]
    h = _point_conv_call(
        xg2T, geoc2, geot2, Wpc2.T, Wpc2[8:11, :], bpc2[None, :], R=256,
        mlp_ws=(Wm1, bm1[None, :], Wm2, bm2[None, :], Wm3, bm3[None, :]),
        interpret=interpret)
    return (h, geoin[:, :N2, :])


# speedup vs baseline: 30.5058x; 30.5058x over previous
import jax
import jax.numpy as jnp
from jax.experimental import pallas as pl
from jax.experimental.pallas import tpu as pltpu

_K = 16
_INF = jnp.inf


def _pc_body(xgT_ref, geoc_ref, geot_ref, wT_ref, wg_ref, b_ref, *rest):
    out_ref = rest[-1]
    mlp = rest[:-1]
    xgT = xgT_ref[0]
    geot = geot_ref[0]
    geoc = geoc_ref[0]
    projT = jnp.dot(wT_ref[...], xgT,
                    preferred_element_type=jnp.float32)
    d = None
    for dim in range(3):
        diff = geoc[:, dim:dim + 1] - geot[dim:dim + 1, :]
        sq = diff * diff
        d = sq if d is None else d + sq
    for _ in range(_K):
        m = jnp.min(d, axis=1, keepdims=True)
        d = jnp.where(d == m, _INF, d)
    sel = d == _INF
    cols = []
    for c in range(8):
        mc = jnp.max(jnp.where(sel, projT[c:c + 1, :], -_INF),
                     axis=1, keepdims=True)
        cols.append(mc)
    mx = jnp.concatenate(cols, axis=1)
    wg = wg_ref[...]
    dotc = (geoc[:, 0:1] * wg[0:1, :] + geoc[:, 1:2] * wg[1:2, :]
            + geoc[:, 2:3] * wg[2:3, :])
    h = jnp.maximum(mx - dotc + b_ref[...], 0.0)
    if mlp:
        wm1, bm1, wm2, bm2, wm3, bm3 = mlp
        h = jnp.maximum(
            jnp.dot(h, wm1[...], preferred_element_type=jnp.float32)
            + bm1[...], 0.0)
        h = jnp.maximum(
            jnp.dot(h, wm2[...], preferred_element_type=jnp.float32)
            + bm2[...], 0.0)
        h = jnp.maximum(
            jnp.dot(h, wm3[...], preferred_element_type=jnp.float32)
            + bm3[...], 0.0)
    out_ref[0] = h


def _full_spec(shape):
    nd = len(shape)
    return pl.BlockSpec(shape, lambda b, r, _n=nd: (0,) * _n)


def _point_conv_call(xgT, geoc, geot, wT, wg, b, R, mlp_ws=()):
    B, _, N = xgT.shape
    Nout = geoc.shape[1]
    Fout = 6 if mlp_ws else 8
    grid = (B, Nout // R)
    in_specs = [
        pl.BlockSpec((1, 11, N), lambda b_, r: (b_, 0, 0)),
        pl.BlockSpec((1, R, 3), lambda b_, r: (b_, r, 0)),
        pl.BlockSpec((1, 3, N), lambda b_, r: (b_, 0, 0)),
        _full_spec(wT.shape),
        _full_spec(wg.shape),
        _full_spec(b.shape),
    ]
    args = [xgT, geoc, geot, wT, wg, b]
    for w in mlp_ws:
        in_specs.append(_full_spec(w.shape))
        args.append(w)
    return pl.pallas_call(
        _pc_body,
        grid=grid,
        in_specs=in_specs,
        out_specs=pl.BlockSpec((1, R, Fout), lambda b_, r: (b_, r, 0)),
        out_shape=jax.ShapeDtypeStruct((B, Nout, Fout), jnp.float32),
        compiler_params=pltpu.CompilerParams(
            dimension_semantics=("parallel", "parallel")),
    )(*args)


def kernel(x, geoin, Wpc1, bpc1, Wpc2, bpc2, Wm1, bm1, Wm2, bm2, Wm3, bm3):
    B, Ns, _ = x.shape
    N1, N2 = Ns // 2, Ns // 4
    xg1T = jnp.concatenate([x, geoin], axis=-1).transpose(0, 2, 1)
    geot1 = geoin.transpose(0, 2, 1)
    geoc1 = geoin[:, :N1, :]
    h1 = _point_conv_call(xg1T, geoc1, geot1, Wpc1.T, Wpc1[8:11, :],
                          bpc1[None, :], R=256)
    xg2T = jnp.concatenate([h1, geoc1], axis=-1).transpose(0, 2, 1)
    geot2 = geoc1.transpose(0, 2, 1)
    geoc2 = geoin[:, :N2, :]
    h = _point_conv_call(
        xg2T, geoc2, geot2, Wpc2.T, Wpc2[8:11, :], bpc2[None, :], R=256,
        mlp_ws=(Wm1, bm1[None, :], Wm2, bm2[None, :], Wm3, bm3[None, :]))
    return (h, geoin[:, :N2, :])
